# baseline (speedup 1.0000x reference)
import functools

import jax
import jax.numpy as jnp
from jax import lax
from jax.experimental import pallas as pl
from jax.experimental.pallas import tpu as pltpu
from jax.experimental.pallas import tpu_sc as plsc

N = 10000
D = 256
H = 128
E = 160000
NS = 16
ROWS = 624
TAIL = N - NS * ROWS
ZR = 48
ECH = 80
ECH2 = 128
DCH = 40
RB = 400


@functools.cache
def _make_deg():
    mesh = plsc.VectorSubcoreMesh(core_axis_name="c", subcore_axis_name="s")
    return pl.kernel(
        _deg_body,
        mesh=mesh,
        out_type=jax.ShapeDtypeStruct((2 * N, H), jnp.float32),
        scratch_types=[
            pltpu.VMEM((E // 32,), jnp.int32),
            pltpu.VMEM((ECH,), jnp.int32),
            pltpu.VMEM((ECH,), jnp.int32),
            pltpu.VMEM((40,), jnp.int32),
            pltpu.VMEM((ECH, H), jnp.float32),
            pltpu.VMEM((ZR, H), jnp.float32),
            pltpu.VMEM_SHARED((N, H), jnp.float32),
            pltpu.SemaphoreType.DMA,
            pltpu.SemaphoreType.DMA,
        ],
    )


def _deg_body(dst_hbm, degp_hbm, dstb, didx0, didx1, didxt, ones_v, zb, acc,
              s0, s1):
    cid = lax.axis_index("c")
    sid = lax.axis_index("s")
    ntile = E // 32

    pltpu.sync_copy(dst_hbm.at[pl.ds(cid * (E // 2) + sid * ntile, ntile)], dstb)

    def fill_ones(i, carry):
        for j in range(H // 16):
            ones_v[i, pl.ds(j * 16, 16)] = jnp.ones((16,), jnp.float32)
        return carry

    lax.fori_loop(0, ECH, fill_ones, 0)

    def fill_zero(i, carry):
        for j in range(H // 16):
            zb[i, pl.ds(j * 16, 16)] = jnp.zeros((16,), jnp.float32)
        return carry

    lax.fori_loop(0, ZR, fill_zero, 0)
    for m in range(ROWS // ZR):
        pltpu.sync_copy(zb, acc.at[pl.ds(sid * ROWS + m * ZR, ZR)])

    @pl.when(sid == NS - 1)
    def _():
        pltpu.sync_copy(zb.at[pl.ds(0, TAIL)], acc.at[pl.ds(NS * ROWS, TAIL)])

    plsc.subcore_barrier()

    nfull = ntile // ECH
    dbufs = (didx0, didx1)
    ssems = (s0, s1)

    def pair(p, carry):
        for k in range(2):
            c = p * 2 + k
            for m in range(ECH // 16):
                dbufs[k][pl.ds(m * 16, 16)] = dstb[pl.ds(c * ECH + m * 16, 16)]
            pltpu.async_copy(ones_v, acc.at[dbufs[k]], ssems[k], add=True)
        for k in range(2):
            pltpu.make_async_copy(ones_v, acc.at[dbufs[k]], ssems[k]).wait()
        return carry

    lax.fori_loop(0, nfull // 2, pair, 0)
    t = nfull * ECH
    didxt[pl.ds(0, 16)] = dstb[pl.ds(t, 16)]
    didxt[pl.ds(16, 16)] = dstb[pl.ds(t + 16, 16)]
    didxt[pl.ds(24, 16)] = dstb[pl.ds(t + 24, 16)]
    pltpu.sync_copy(ones_v.at[pl.ds(0, 40)], acc.at[didxt], add=True)
    plsc.subcore_barrier()
    pltpu.sync_copy(acc.at[pl.ds(sid * ROWS, ROWS)],
                    degp_hbm.at[pl.ds(cid * N + sid * ROWS, ROWS)])

    @pl.when(sid == NS - 1)
    def _():
        pltpu.sync_copy(acc.at[pl.ds(NS * ROWS, TAIL)],
                        degp_hbm.at[pl.ds(cid * N + NS * ROWS, TAIL)])


@functools.cache
def _make_agg():
    mesh = plsc.VectorSubcoreMesh(core_axis_name="c", subcore_axis_name="s")
    return pl.kernel(
        _agg_body,
        mesh=mesh,
        out_type=jax.ShapeDtypeStruct((2 * N, H), jnp.float32),
        scratch_types=[
            pltpu.VMEM((E // NS,), jnp.int32),
            pltpu.VMEM((ECH2,), jnp.int32),
            pltpu.VMEM((ECH2,), jnp.int32),
            pltpu.VMEM((16,), jnp.int32),
            pltpu.VMEM((ECH2, H), jnp.float32),
            pltpu.VMEM((ECH2, H), jnp.float32),
            pltpu.VMEM_SHARED((N, H), jnp.float32),
            pltpu.SemaphoreType.DMA,
            pltpu.SemaphoreType.DMA,
            pltpu.SemaphoreType.DMA,
            pltpu.SemaphoreType.DMA,
            pltpu.SemaphoreType.DMA,
            pltpu.SemaphoreType.DMA,
        ],
    )


def _agg_body(y_hbm, src_hbm, dst_hbm, s_hbm, srcb, didx0, didx1, didxt,
              r0, r1, acc, g0, g1, d0, d1, s0, s1):
    cid = lax.axis_index("c")
    sid = lax.axis_index("s")
    bufs = (r0, r1)
    gsems = (g0, g1)
    dbufs = (didx0, didx1)
    dsems = (d0, d1)
    ssems = (s0, s1)
    ntile = E // NS
    nch = (ntile // ECH2)
    base = sid * ntile

    pltpu.sync_copy(src_hbm.at[pl.ds(base, ntile)], srcb)
    off = cid * N

    def adjf(i, carry):
        srcb[pl.ds(i * 16, 16)] = srcb[pl.ds(i * 16, 16)] + off
        return carry

    lax.fori_loop(0, ntile // 16, adjf, 0)

    def fill_zero(i, carry):
        for j in range(H // 16):
            r0[i, pl.ds(j * 16, 16)] = jnp.zeros((16,), jnp.float32)
        return carry

    lax.fori_loop(0, ECH2, fill_zero, 0)
    for m in range(4):
        pltpu.sync_copy(r0, acc.at[pl.ds(sid * ROWS + m * ECH2, ECH2)])
    pltpu.sync_copy(r0.at[pl.ds(0, ROWS - 4 * ECH2)],
                    acc.at[pl.ds(sid * ROWS + 4 * ECH2, ROWS - 4 * ECH2)])

    @pl.when(sid == NS - 1)
    def _():
        pltpu.sync_copy(r0.at[pl.ds(0, TAIL)], acc.at[pl.ds(NS * ROWS, TAIL)])

    plsc.subcore_barrier()

    def fetch(c, k):
        pltpu.async_copy(dst_hbm.at[pl.ds(base + c * ECH2, ECH2)],
                         dbufs[k], dsems[k])
        pltpu.async_copy(y_hbm.at[srcb.at[pl.ds(c * ECH2, ECH2)]],
                         bufs[k], gsems[k])

    def waitboth(k):
        pltpu.make_async_copy(dst_hbm.at[pl.ds(0, ECH2)],
                              dbufs[k], dsems[k]).wait()
        pltpu.make_async_copy(y_hbm.at[srcb.at[pl.ds(0, ECH2)]],
                              bufs[k], gsems[k]).wait()

    for k in range(2):
        fetch(k, k)

    def grp(p, carry):
        for k in range(2):
            waitboth(k)
            pltpu.async_copy(bufs[k], acc.at[dbufs[k]], ssems[k], add=True)
        for k in range(2):
            c = p * 2 + k
            pltpu.make_async_copy(bufs[k], acc.at[dbufs[k]], ssems[k]).wait()

            @pl.when(c + 2 < nch)
            def _():
                fetch(c + 2, k)

        return carry

    lax.fori_loop(0, nch // 2, grp, 0)
    pltpu.sync_copy(dst_hbm.at[pl.ds(base + nch * ECH2, 16)], didxt)
    pltpu.async_copy(y_hbm.at[srcb.at[pl.ds(nch * ECH2, 16)]],
                     r0.at[pl.ds(0, 16)], g0).wait()
    pltpu.sync_copy(r0.at[pl.ds(0, 16)], acc.at[didxt], add=True)
    plsc.subcore_barrier()
    pltpu.sync_copy(acc.at[pl.ds(sid * ROWS, ROWS)],
                    s_hbm.at[pl.ds(cid * N + sid * ROWS, ROWS)])

    @pl.when(sid == NS - 1)
    def _():
        pltpu.sync_copy(acc.at[pl.ds(NS * ROWS, TAIL)],
                        s_hbm.at[pl.ds(cid * N + NS * ROWS, TAIL)])


def _dinv_block(degp_ref):
    deg = degp_ref[0, :, 0:1] + degp_ref[1, :, 0:1] + 1.0
    return lax.rsqrt(deg)


def _dense1_body(x_ref, w_ref, degp_ref, y_ref):
    dinv = _dinv_block(degp_ref)
    h = jnp.dot(x_ref[...], w_ref[...], preferred_element_type=jnp.float32)
    y = h * dinv
    y_ref[0] = y[:, :H]
    y_ref[1] = y[:, H:]


def _dense2_body(s_ref, y_ref, degp_ref, w_ref, b_ref, o_ref):
    dinv = _dinv_block(degp_ref)
    r0 = jnp.maximum((s_ref[0] + y_ref[0]) * dinv + b_ref[0:1, :], 0.0)
    r1 = jnp.maximum((s_ref[1] + y_ref[1]) * dinv + b_ref[1:2, :], 0.0)
    h2 = (jnp.dot(r0, w_ref[0], preferred_element_type=jnp.float32)
          + jnp.dot(r1, w_ref[1], preferred_element_type=jnp.float32))
    y2 = h2 * dinv
    o_ref[0] = y2[:, :H]
    o_ref[1] = y2[:, H:]


def _dense3_body(s_ref, y_ref, degp_ref, x_ref, wg_ref, b_ref, bg_ref, o_ref):
    dinv = _dinv_block(degp_ref)
    r0 = jnp.maximum((s_ref[0] + y_ref[0]) * dinv + b_ref[0:1, :], 0.0)
    r1 = jnp.maximum((s_ref[1] + y_ref[1]) * dinv + b_ref[1:2, :], 0.0)
    xb = x_ref[...]
    h = jnp.concatenate([r0, r1], axis=1) + xb
    gi = (jnp.dot(h, wg_ref[0], preferred_element_type=jnp.float32)
          + jnp.dot(xb, wg_ref[1], preferred_element_type=jnp.float32)
          + bg_ref[...])
    o_ref[...] = h * jax.nn.sigmoid(gi)


_GRID = N // RB

_dense1 = pl.pallas_call(
    _dense1_body,
    grid=(_GRID,),
    in_specs=[
        pl.BlockSpec((RB, D), lambda i: (i, 0)),
        pl.BlockSpec((D, D), lambda i: (0, 0)),
        pl.BlockSpec((2, RB, 16), lambda i: (0, i, 0)),
    ],
    out_specs=pl.BlockSpec((2, RB, H), lambda i: (0, i, 0)),
    out_shape=jax.ShapeDtypeStruct((2, N, H), jnp.float32),
)

_dense2 = pl.pallas_call(
    _dense2_body,
    grid=(_GRID,),
    in_specs=[
        pl.BlockSpec((2, RB, H), lambda i: (0, i, 0)),
        pl.BlockSpec((2, RB, H), lambda i: (0, i, 0)),
        pl.BlockSpec((2, RB, 16), lambda i: (0, i, 0)),
        pl.BlockSpec((2, H, D), lambda i: (0, 0, 0)),
        pl.BlockSpec((2, H), lambda i: (0, 0)),
    ],
    out_specs=pl.BlockSpec((2, RB, H), lambda i: (0, i, 0)),
    out_shape=jax.ShapeDtypeStruct((2, N, H), jnp.float32),
)

_dense3 = pl.pallas_call(
    _dense3_body,
    grid=(_GRID,),
    in_specs=[
        pl.BlockSpec((2, RB, H), lambda i: (0, i, 0)),
        pl.BlockSpec((2, RB, H), lambda i: (0, i, 0)),
        pl.BlockSpec((2, RB, 16), lambda i: (0, i, 0)),
        pl.BlockSpec((RB, D), lambda i: (i, 0)),
        pl.BlockSpec((2, D, D), lambda i: (0, 0, 0)),
        pl.BlockSpec((2, H), lambda i: (0, 0)),
        pl.BlockSpec((1, D), lambda i: (0, 0)),
    ],
    out_specs=pl.BlockSpec((RB, D), lambda i: (i, 0)),
    out_shape=jax.ShapeDtypeStruct((N, D), jnp.float32),
)


def kernel(x, edge_index, W1, b1, W2, b2, Wg, bg):
    ei = edge_index.astype(jnp.int32)
    src = ei[0]
    dst = ei[1]
    deg_call = _make_deg()
    agg_call = _make_agg()
    degp = deg_call(dst)[:, :16].reshape(2, N, 16)
    y1 = _dense1(x, W1, degp)
    s1 = agg_call(y1.reshape(2 * N, H), src, dst).reshape(2, N, H)
    y2 = _dense2(s1, y1, degp, W2.reshape(2, H, D), b1.reshape(2, H))
    s2 = agg_call(y2.reshape(2 * N, H), src, dst).reshape(2, N, H)
    out = _dense3(s2, y2, degp, x, Wg.reshape(2, D, D), b2.reshape(2, H),
                  bg.reshape(1, D))
    return out


# speedup vs baseline: 1.2028x; 1.2028x over previous
import functools

import jax
import jax.numpy as jnp
from jax import lax
from jax.experimental import pallas as pl
from jax.experimental.pallas import tpu as pltpu
from jax.experimental.pallas import tpu_sc as plsc

N = 10000
D = 256
H = 128
E = 160000
NS = 16
ROWS = 624
TAIL = N - NS * ROWS
ZR = 48
ECH = 80
ECH2 = 128
DCH = 40
RB = 400


@functools.cache
def _make_deg():
    mesh = plsc.VectorSubcoreMesh(core_axis_name="c", subcore_axis_name="s")
    return pl.kernel(
        _deg_body,
        mesh=mesh,
        out_type=jax.ShapeDtypeStruct((2 * N, H), jnp.float32),
        scratch_types=[
            pltpu.VMEM((E // 32,), jnp.int32),
            pltpu.VMEM((ECH,), jnp.int32),
            pltpu.VMEM((ECH,), jnp.int32),
            pltpu.VMEM((40,), jnp.int32),
            pltpu.VMEM((ECH, H), jnp.float32),
            pltpu.VMEM((ZR, H), jnp.float32),
            pltpu.VMEM_SHARED((N, H), jnp.float32),
        ],
    )


def _deg_body(dst_hbm, degp_hbm, dstb, didx0, didx1, didxt, ones_v, zb, acc):
    cid = lax.axis_index("c")
    sid = lax.axis_index("s")
    ntile = E // 32

    pltpu.sync_copy(dst_hbm.at[pl.ds(cid * (E // 2) + sid * ntile, ntile)], dstb)

    def fill_ones(i, carry):
        for j in range(H // 16):
            ones_v[i, pl.ds(j * 16, 16)] = jnp.ones((16,), jnp.float32)
        return carry

    lax.fori_loop(0, ECH, fill_ones, 0)

    def fill_zero(i, carry):
        for j in range(H // 16):
            zb[i, pl.ds(j * 16, 16)] = jnp.zeros((16,), jnp.float32)
        return carry

    lax.fori_loop(0, ZR, fill_zero, 0)
    for m in range(ROWS // ZR):
        pltpu.sync_copy(zb, acc.at[pl.ds(sid * ROWS + m * ZR, ZR)])

    @pl.when(sid == NS - 1)
    def _():
        pltpu.sync_copy(zb.at[pl.ds(0, TAIL)], acc.at[pl.ds(NS * ROWS, TAIL)])

    plsc.subcore_barrier()

    nfull = ntile // ECH
    dbufs = (didx0, didx1)

    def pair(p, carry):
        for k in range(2):
            c = p * 2 + k
            for m in range(ECH // 16):
                dbufs[k][pl.ds(m * 16, 16)] = dstb[pl.ds(c * ECH + m * 16, 16)]
            pltpu.sync_copy(ones_v, acc.at[dbufs[k]], add=True)
        return carry

    lax.fori_loop(0, nfull // 2, pair, 0)
    t = nfull * ECH
    didxt[pl.ds(0, 16)] = dstb[pl.ds(t, 16)]
    didxt[pl.ds(16, 16)] = dstb[pl.ds(t + 16, 16)]
    didxt[pl.ds(24, 16)] = dstb[pl.ds(t + 24, 16)]
    pltpu.sync_copy(ones_v.at[pl.ds(0, 40)], acc.at[didxt], add=True)
    plsc.subcore_barrier()
    pltpu.sync_copy(acc.at[pl.ds(sid * ROWS, ROWS)],
                    degp_hbm.at[pl.ds(cid * N + sid * ROWS, ROWS)])

    @pl.when(sid == NS - 1)
    def _():
        pltpu.sync_copy(acc.at[pl.ds(NS * ROWS, TAIL)],
                        degp_hbm.at[pl.ds(cid * N + NS * ROWS, TAIL)])


@functools.cache
def _make_agg():
    mesh = plsc.VectorSubcoreMesh(core_axis_name="c", subcore_axis_name="s")
    return pl.kernel(
        _agg_body,
        mesh=mesh,
        out_type=jax.ShapeDtypeStruct((2 * N, H), jnp.float32),
        scratch_types=[
            pltpu.VMEM((E // NS,), jnp.int32),
            pltpu.VMEM((ECH2,), jnp.int32),
            pltpu.VMEM((ECH2,), jnp.int32),
            pltpu.VMEM((16,), jnp.int32),
            pltpu.VMEM((ECH2, H), jnp.float32),
            pltpu.VMEM((ECH2, H), jnp.float32),
            pltpu.VMEM_SHARED((N, H), jnp.float32),
            pltpu.SemaphoreType.DMA,
            pltpu.SemaphoreType.DMA,
            pltpu.SemaphoreType.DMA,
            pltpu.SemaphoreType.DMA,
        ],
    )


def _agg_body(y_hbm, src_hbm, dst_hbm, s_hbm, srcb, didx0, didx1, didxt,
              r0, r1, acc, g0, g1, d0, d1):
    cid = lax.axis_index("c")
    sid = lax.axis_index("s")
    bufs = (r0, r1)
    gsems = (g0, g1)
    dbufs = (didx0, didx1)
    dsems = (d0, d1)
    ntile = E // NS
    nch = (ntile // ECH2)
    base = sid * ntile

    pltpu.sync_copy(src_hbm.at[pl.ds(base, ntile)], srcb)
    off = cid * N

    def adjf(i, carry):
        srcb[pl.ds(i * 16, 16)] = srcb[pl.ds(i * 16, 16)] + off
        return carry

    lax.fori_loop(0, ntile // 16, adjf, 0)

    def fill_zero(i, carry):
        for j in range(H // 16):
            r0[i, pl.ds(j * 16, 16)] = jnp.zeros((16,), jnp.float32)
        return carry

    lax.fori_loop(0, ECH2, fill_zero, 0)
    for m in range(4):
        pltpu.sync_copy(r0, acc.at[pl.ds(sid * ROWS + m * ECH2, ECH2)])
    pltpu.sync_copy(r0.at[pl.ds(0, ROWS - 4 * ECH2)],
                    acc.at[pl.ds(sid * ROWS + 4 * ECH2, ROWS - 4 * ECH2)])

    @pl.when(sid == NS - 1)
    def _():
        pltpu.sync_copy(r0.at[pl.ds(0, TAIL)], acc.at[pl.ds(NS * ROWS, TAIL)])

    plsc.subcore_barrier()

    def fetch(c, k):
        pltpu.async_copy(dst_hbm.at[pl.ds(base + c * ECH2, ECH2)],
                         dbufs[k], dsems[k])
        pltpu.async_copy(y_hbm.at[srcb.at[pl.ds(c * ECH2, ECH2)]],
                         bufs[k], gsems[k])

    def waitboth(k):
        pltpu.make_async_copy(dst_hbm.at[pl.ds(0, ECH2)],
                              dbufs[k], dsems[k]).wait()
        pltpu.make_async_copy(y_hbm.at[srcb.at[pl.ds(0, ECH2)]],
                              bufs[k], gsems[k]).wait()

    for k in range(2):
        fetch(k, k)

    def grp(p, carry):
        for k in range(2):
            c = p * 2 + k
            waitboth(k)
            pltpu.sync_copy(bufs[k], acc.at[dbufs[k]], add=True)

            @pl.when(c + 2 < nch)
            def _():
                fetch(c + 2, k)

        return carry

    lax.fori_loop(0, nch // 2, grp, 0)
    pltpu.sync_copy(dst_hbm.at[pl.ds(base + nch * ECH2, 16)], didxt)
    pltpu.async_copy(y_hbm.at[srcb.at[pl.ds(nch * ECH2, 16)]],
                     r0.at[pl.ds(0, 16)], g0).wait()
    pltpu.sync_copy(r0.at[pl.ds(0, 16)], acc.at[didxt], add=True)
    plsc.subcore_barrier()
    pltpu.sync_copy(acc.at[pl.ds(sid * ROWS, ROWS)],
                    s_hbm.at[pl.ds(cid * N + sid * ROWS, ROWS)])

    @pl.when(sid == NS - 1)
    def _():
        pltpu.sync_copy(acc.at[pl.ds(NS * ROWS, TAIL)],
                        s_hbm.at[pl.ds(cid * N + NS * ROWS, TAIL)])


def _dinv_block(degp_ref):
    deg = degp_ref[0, :, 0:1] + degp_ref[1, :, 0:1] + 1.0
    return lax.rsqrt(deg)


def _dense1_body(x_ref, w_ref, degp_ref, y_ref):
    dinv = _dinv_block(degp_ref)
    h = jnp.dot(x_ref[...], w_ref[...], preferred_element_type=jnp.float32)
    y = h * dinv
    y_ref[0] = y[:, :H]
    y_ref[1] = y[:, H:]


def _dense2_body(s_ref, y_ref, degp_ref, w_ref, b_ref, o_ref):
    dinv = _dinv_block(degp_ref)
    r0 = jnp.maximum((s_ref[0] + y_ref[0]) * dinv + b_ref[0:1, :], 0.0)
    r1 = jnp.maximum((s_ref[1] + y_ref[1]) * dinv + b_ref[1:2, :], 0.0)
    h2 = (jnp.dot(r0, w_ref[0], preferred_element_type=jnp.float32)
          + jnp.dot(r1, w_ref[1], preferred_element_type=jnp.float32))
    y2 = h2 * dinv
    o_ref[0] = y2[:, :H]
    o_ref[1] = y2[:, H:]


def _dense3_body(s_ref, y_ref, degp_ref, x_ref, wg_ref, b_ref, bg_ref, o_ref):
    dinv = _dinv_block(degp_ref)
    r0 = jnp.maximum((s_ref[0] + y_ref[0]) * dinv + b_ref[0:1, :], 0.0)
    r1 = jnp.maximum((s_ref[1] + y_ref[1]) * dinv + b_ref[1:2, :], 0.0)
    xb = x_ref[...]
    h = jnp.concatenate([r0, r1], axis=1) + xb
    gi = (jnp.dot(h, wg_ref[0], preferred_element_type=jnp.float32)
          + jnp.dot(xb, wg_ref[1], preferred_element_type=jnp.float32)
          + bg_ref[...])
    o_ref[...] = h * jax.nn.sigmoid(gi)


_GRID = N // RB

_dense1 = pl.pallas_call(
    _dense1_body,
    grid=(_GRID,),
    in_specs=[
        pl.BlockSpec((RB, D), lambda i: (i, 0)),
        pl.BlockSpec((D, D), lambda i: (0, 0)),
        pl.BlockSpec((2, RB, 16), lambda i: (0, i, 0)),
    ],
    out_specs=pl.BlockSpec((2, RB, H), lambda i: (0, i, 0)),
    out_shape=jax.ShapeDtypeStruct((2, N, H), jnp.float32),
)

_dense2 = pl.pallas_call(
    _dense2_body,
    grid=(_GRID,),
    in_specs=[
        pl.BlockSpec((2, RB, H), lambda i: (0, i, 0)),
        pl.BlockSpec((2, RB, H), lambda i: (0, i, 0)),
        pl.BlockSpec((2, RB, 16), lambda i: (0, i, 0)),
        pl.BlockSpec((2, H, D), lambda i: (0, 0, 0)),
        pl.BlockSpec((2, H), lambda i: (0, 0)),
    ],
    out_specs=pl.BlockSpec((2, RB, H), lambda i: (0, i, 0)),
    out_shape=jax.ShapeDtypeStruct((2, N, H), jnp.float32),
)

_dense3 = pl.pallas_call(
    _dense3_body,
    grid=(_GRID,),
    in_specs=[
        pl.BlockSpec((2, RB, H), lambda i: (0, i, 0)),
        pl.BlockSpec((2, RB, H), lambda i: (0, i, 0)),
        pl.BlockSpec((2, RB, 16), lambda i: (0, i, 0)),
        pl.BlockSpec((RB, D), lambda i: (i, 0)),
        pl.BlockSpec((2, D, D), lambda i: (0, 0, 0)),
        pl.BlockSpec((2, H), lambda i: (0, 0)),
        pl.BlockSpec((1, D), lambda i: (0, 0)),
    ],
    out_specs=pl.BlockSpec((RB, D), lambda i: (i, 0)),
    out_shape=jax.ShapeDtypeStruct((N, D), jnp.float32),
)


def kernel(x, edge_index, W1, b1, W2, b2, Wg, bg):
    ei = edge_index.astype(jnp.int32)
    src = ei[0]
    dst = ei[1]
    deg_call = _make_deg()
    agg_call = _make_agg()
    degp = deg_call(dst)[:, :16].reshape(2, N, 16)
    y1 = _dense1(x, W1, degp)
    s1 = agg_call(y1.reshape(2 * N, H), src, dst).reshape(2, N, H)
    y2 = _dense2(s1, y1, degp, W2.reshape(2, H, D), b1.reshape(2, H))
    s2 = agg_call(y2.reshape(2 * N, H), src, dst).reshape(2, N, H)
    out = _dense3(s2, y2, degp, x, Wg.reshape(2, D, D), b2.reshape(2, H),
                  bg.reshape(1, D))
    return out


# speedup vs baseline: 1.2131x; 1.0086x over previous
import functools

import jax
import jax.numpy as jnp
from jax import lax
from jax.experimental import pallas as pl
from jax.experimental.pallas import tpu as pltpu
from jax.experimental.pallas import tpu_sc as plsc

N = 10000
D = 256
H = 128
E = 160000
NS = 16
ROWS = 624
TAIL = N - NS * ROWS
ZR = 48
ECH = 80
ECH2 = 128
DCH = 40
RB = 400


@functools.cache
def _make_deg():
    mesh = plsc.VectorSubcoreMesh(core_axis_name="c", subcore_axis_name="s")
    return pl.kernel(
        _deg_body,
        mesh=mesh,
        out_type=jax.ShapeDtypeStruct((2 * N, H), jnp.float32),
        scratch_types=[
            pltpu.VMEM((5120,), jnp.int32),
            pltpu.VMEM((ECH2,), jnp.int32),
            pltpu.VMEM((ECH2, H), jnp.float32),
            pltpu.VMEM((ZR, H), jnp.float32),
            pltpu.VMEM_SHARED((N, H), jnp.float32),
        ],
    )


def _deg_body(dst_hbm, degp_hbm, dstb, didx, ones_v, zb, acc):
    cid = lax.axis_index("c")
    sid = lax.axis_index("s")
    base = cid * (E // 2) + sid * 4992

    pltpu.sync_copy(dst_hbm.at[pl.ds(base, 5120)], dstb)

    def fill_ones(i, carry):
        for j in range(H // 16):
            ones_v[i, pl.ds(j * 16, 16)] = jnp.ones((16,), jnp.float32)
        return carry

    lax.fori_loop(0, ECH2, fill_ones, 0)

    def fill_zero(i, carry):
        for j in range(H // 16):
            zb[i, pl.ds(j * 16, 16)] = jnp.zeros((16,), jnp.float32)
        return carry

    lax.fori_loop(0, ZR, fill_zero, 0)
    for m in range(ROWS // ZR):
        pltpu.sync_copy(zb, acc.at[pl.ds(sid * ROWS + m * ZR, ZR)])

    @pl.when(sid == NS - 1)
    def _():
        pltpu.sync_copy(zb.at[pl.ds(0, TAIL)], acc.at[pl.ds(NS * ROWS, TAIL)])

    plsc.subcore_barrier()

    nfull = 39 + jnp.where(sid == NS - 1, 1, 0)

    def step(c, carry):
        for m in range(ECH2 // 16):
            didx[pl.ds(m * 16, 16)] = dstb[pl.ds(c * ECH2 + m * 16, 16)]
        pltpu.sync_copy(ones_v, acc.at[didx], add=True)
        return carry

    lax.fori_loop(0, nfull, step, 0)
    plsc.subcore_barrier()
    pltpu.sync_copy(acc.at[pl.ds(sid * ROWS, ROWS)],
                    degp_hbm.at[pl.ds(cid * N + sid * ROWS, ROWS)])

    @pl.when(sid == NS - 1)
    def _():
        pltpu.sync_copy(acc.at[pl.ds(NS * ROWS, TAIL)],
                        degp_hbm.at[pl.ds(cid * N + NS * ROWS, TAIL)])


@functools.cache
def _make_agg():
    mesh = plsc.VectorSubcoreMesh(core_axis_name="c", subcore_axis_name="s")
    return pl.kernel(
        _agg_body,
        mesh=mesh,
        out_type=jax.ShapeDtypeStruct((2 * N, H), jnp.float32),
        scratch_types=[
            pltpu.VMEM((10240,), jnp.int32),
            pltpu.VMEM((ECH2,), jnp.int32),
            pltpu.VMEM((ECH2,), jnp.int32),
            pltpu.VMEM((ECH2, H), jnp.float32),
            pltpu.VMEM((ECH2, H), jnp.float32),
            pltpu.VMEM_SHARED((N, H), jnp.float32),
            pltpu.SemaphoreType.DMA,
            pltpu.SemaphoreType.DMA,
            pltpu.SemaphoreType.DMA,
            pltpu.SemaphoreType.DMA,
        ],
    )


def _agg_body(y_hbm, src_hbm, dst_hbm, s_hbm, srcb, didx0, didx1,
              r0, r1, acc, g0, g1, d0, d1):
    cid = lax.axis_index("c")
    sid = lax.axis_index("s")
    bufs = (r0, r1)
    gsems = (g0, g1)
    dbufs = (didx0, didx1)
    dsems = (d0, d1)
    nch = 78 + jnp.where(sid == NS - 1, 2, 0)
    base = sid * 9984

    pltpu.sync_copy(src_hbm.at[pl.ds(base, 10240)], srcb)
    off = cid * N

    def adjf(i, carry):
        srcb[pl.ds(i * 16, 16)] = srcb[pl.ds(i * 16, 16)] + off
        return carry

    lax.fori_loop(0, 10240 // 16, adjf, 0)

    def fill_zero(i, carry):
        for j in range(H // 16):
            r0[i, pl.ds(j * 16, 16)] = jnp.zeros((16,), jnp.float32)
        return carry

    lax.fori_loop(0, ECH2, fill_zero, 0)
    for m in range(4):
        pltpu.sync_copy(r0, acc.at[pl.ds(sid * ROWS + m * ECH2, ECH2)])
    pltpu.sync_copy(r0.at[pl.ds(0, ROWS - 4 * ECH2)],
                    acc.at[pl.ds(sid * ROWS + 4 * ECH2, ROWS - 4 * ECH2)])

    @pl.when(sid == NS - 1)
    def _():
        pltpu.sync_copy(r0.at[pl.ds(0, TAIL)], acc.at[pl.ds(NS * ROWS, TAIL)])

    plsc.subcore_barrier()

    def fetch(c, k):
        pltpu.async_copy(dst_hbm.at[pl.ds(base + c * ECH2, ECH2)],
                         dbufs[k], dsems[k])
        pltpu.async_copy(y_hbm.at[srcb.at[pl.ds(c * ECH2, ECH2)]],
                         bufs[k], gsems[k])

    def waitboth(k):
        pltpu.make_async_copy(dst_hbm.at[pl.ds(0, ECH2)],
                              dbufs[k], dsems[k]).wait()
        pltpu.make_async_copy(y_hbm.at[srcb.at[pl.ds(0, ECH2)]],
                              bufs[k], gsems[k]).wait()

    for k in range(2):
        fetch(k, k)

    def grp(p, carry):
        for k in range(2):
            c = p * 2 + k
            waitboth(k)
            pltpu.sync_copy(bufs[k], acc.at[dbufs[k]], add=True)

            @pl.when(c + 2 < nch)
            def _():
                fetch(c + 2, k)

        return carry

    lax.fori_loop(0, nch // 2, grp, 0)
    plsc.subcore_barrier()
    pltpu.sync_copy(acc.at[pl.ds(sid * ROWS, ROWS)],
                    s_hbm.at[pl.ds(cid * N + sid * ROWS, ROWS)])

    @pl.when(sid == NS - 1)
    def _():
        pltpu.sync_copy(acc.at[pl.ds(NS * ROWS, TAIL)],
                        s_hbm.at[pl.ds(cid * N + NS * ROWS, TAIL)])


def _dinv_block(degp_ref):
    deg = degp_ref[0, :, 0:1] + degp_ref[1, :, 0:1] + 1.0
    return lax.rsqrt(deg)


def _dense1_body(x_ref, w_ref, degp_ref, y_ref):
    dinv = _dinv_block(degp_ref)
    h = jnp.dot(x_ref[...], w_ref[...], preferred_element_type=jnp.float32)
    y = h * dinv
    y_ref[0] = y[:, :H]
    y_ref[1] = y[:, H:]


def _dense2_body(s_ref, y_ref, degp_ref, w_ref, b_ref, o_ref):
    dinv = _dinv_block(degp_ref)
    r0 = jnp.maximum((s_ref[0] + y_ref[0]) * dinv + b_ref[0:1, :], 0.0)
    r1 = jnp.maximum((s_ref[1] + y_ref[1]) * dinv + b_ref[1:2, :], 0.0)
    h2 = (jnp.dot(r0, w_ref[0], preferred_element_type=jnp.float32)
          + jnp.dot(r1, w_ref[1], preferred_element_type=jnp.float32))
    y2 = h2 * dinv
    o_ref[0] = y2[:, :H]
    o_ref[1] = y2[:, H:]


def _dense3_body(s_ref, y_ref, degp_ref, x_ref, wg_ref, b_ref, bg_ref, o_ref):
    dinv = _dinv_block(degp_ref)
    r0 = jnp.maximum((s_ref[0] + y_ref[0]) * dinv + b_ref[0:1, :], 0.0)
    r1 = jnp.maximum((s_ref[1] + y_ref[1]) * dinv + b_ref[1:2, :], 0.0)
    xb = x_ref[...]
    h = jnp.concatenate([r0, r1], axis=1) + xb
    gi = (jnp.dot(h, wg_ref[0], preferred_element_type=jnp.float32)
          + jnp.dot(xb, wg_ref[1], preferred_element_type=jnp.float32)
          + bg_ref[...])
    o_ref[...] = h * jax.nn.sigmoid(gi)


_GRID = N // RB

_dense1 = pl.pallas_call(
    _dense1_body,
    grid=(_GRID,),
    in_specs=[
        pl.BlockSpec((RB, D), lambda i: (i, 0)),
        pl.BlockSpec((D, D), lambda i: (0, 0)),
        pl.BlockSpec((2, RB, 16), lambda i: (0, i, 0)),
    ],
    out_specs=pl.BlockSpec((2, RB, H), lambda i: (0, i, 0)),
    out_shape=jax.ShapeDtypeStruct((2, N, H), jnp.float32),
)

_dense2 = pl.pallas_call(
    _dense2_body,
    grid=(_GRID,),
    in_specs=[
        pl.BlockSpec((2, RB, H), lambda i: (0, i, 0)),
        pl.BlockSpec((2, RB, H), lambda i: (0, i, 0)),
        pl.BlockSpec((2, RB, 16), lambda i: (0, i, 0)),
        pl.BlockSpec((2, H, D), lambda i: (0, 0, 0)),
        pl.BlockSpec((2, H), lambda i: (0, 0)),
    ],
    out_specs=pl.BlockSpec((2, RB, H), lambda i: (0, i, 0)),
    out_shape=jax.ShapeDtypeStruct((2, N, H), jnp.float32),
)

_dense3 = pl.pallas_call(
    _dense3_body,
    grid=(_GRID,),
    in_specs=[
        pl.BlockSpec((2, RB, H), lambda i: (0, i, 0)),
        pl.BlockSpec((2, RB, H), lambda i: (0, i, 0)),
        pl.BlockSpec((2, RB, 16), lambda i: (0, i, 0)),
        pl.BlockSpec((RB, D), lambda i: (i, 0)),
        pl.BlockSpec((2, D, D), lambda i: (0, 0, 0)),
        pl.BlockSpec((2, H), lambda i: (0, 0)),
        pl.BlockSpec((1, D), lambda i: (0, 0)),
    ],
    out_specs=pl.BlockSpec((RB, D), lambda i: (i, 0)),
    out_shape=jax.ShapeDtypeStruct((N, D), jnp.float32),
)


def kernel(x, edge_index, W1, b1, W2, b2, Wg, bg):
    ei = edge_index.astype(jnp.int32)
    src = ei[0]
    dst = ei[1]
    deg_call = _make_deg()
    agg_call = _make_agg()
    degp = deg_call(dst)[:, :16].reshape(2, N, 16)
    y1 = _dense1(x, W1, degp)
    s1 = agg_call(y1.reshape(2 * N, H), src, dst).reshape(2, N, H)
    y2 = _dense2(s1, y1, degp, W2.reshape(2, H, D), b1.reshape(2, H))
    s2 = agg_call(y2.reshape(2 * N, H), src, dst).reshape(2, N, H)
    out = _dense3(s2, y2, degp, x, Wg.reshape(2, D, D), b2.reshape(2, H),
                  bg.reshape(1, D))
    return out
